# baseline (speedup 1.0000x reference)
import functools

import jax
import jax.numpy as jnp
from jax import lax
from jax.experimental import pallas as pl
from jax.experimental.pallas import tpu as pltpu
from jax.experimental.pallas import tpu_sc as plsc

N = 10000
E = 320000
H = 128
L = 4
NF = 3
K = 3
IN_DIM = 128

SC_CORES = 2
SC_TILES = 16
NWORK = SC_CORES * SC_TILES
CHUNK = 128
CH = 80
EPT = CH * CHUNK
EP = NWORK * EPT
NP = 10112
ROWS_PT = NP // SC_TILES
NFULL = ROWS_PT // CHUNK
NREM = ROWS_PT - NFULL * CHUNK

_f32 = jnp.float32
_mesh = plsc.VectorSubcoreMesh(
    core_axis_name="c", subcore_axis_name="s",
    num_cores=SC_CORES, num_subcores=SC_TILES)


def _zero_my_rows(zrow_hbm, wbuf, acc_sh, base):
    pltpu.sync_copy(zrow_hbm, wbuf)

    @pl.loop(0, NFULL)
    def _(i):
        pltpu.sync_copy(wbuf, acc_sh.at[pl.ds(base + i * CHUNK, CHUNK)])

    pltpu.sync_copy(wbuf.at[pl.ds(0, NREM)],
                    acc_sh.at[pl.ds(base + NFULL * CHUNK, NREM)])


def _writeback_my_rows(acc_sh, wbuf, p_hbm, c, base):
    @pl.loop(0, NFULL)
    def _(i):
        pltpu.sync_copy(acc_sh.at[pl.ds(base + i * CHUNK, CHUNK)], wbuf)
        pltpu.sync_copy(wbuf, p_hbm.at[c, pl.ds(base + i * CHUNK, CHUNK)])

    pltpu.sync_copy(acc_sh.at[pl.ds(base + NFULL * CHUNK, NREM)],
                    wbuf.at[pl.ds(0, NREM)])
    pltpu.sync_copy(wbuf.at[pl.ds(0, NREM)],
                    p_hbm.at[c, pl.ds(base + NFULL * CHUNK, NREM)])


HALF = CH // 2


@functools.partial(
    pl.kernel,
    out_type=jax.ShapeDtypeStruct((SC_CORES, NP, H), _f32),
    mesh=_mesh,
    scratch_types=[
        pltpu.VMEM((HALF, CHUNK), jnp.int32),
        pltpu.VMEM((HALF, CHUNK), jnp.int32),
        pltpu.VMEM((CHUNK, H), _f32),
        pltpu.VMEM((CHUNK, H), _f32),
        pltpu.VMEM_SHARED((NP, H), _f32),
        pltpu.SemaphoreType.DMA,
        pltpu.SemaphoreType.DMA,
        pltpu.SemaphoreType.DMA,
        pltpu.SemaphoreType.DMA,
    ],
)
def _sc_prop(u_hbm, srcr_hbm, dstr_hbm, zrow_hbm, p_hbm,
             src_v, dst_v, gbuf0, gbuf1, acc_sh, gsem0, gsem1, ssem0, ssem1):
    c = lax.axis_index("c")
    s = lax.axis_index("s")
    wid = s * SC_CORES + c
    base = s * ROWS_PT
    _zero_my_rows(zrow_hbm, gbuf0, acc_sh, base)
    plsc.subcore_barrier()

    for hh in range(CH // HALF):
        pltpu.sync_copy(srcr_hbm.at[wid, pl.ds(hh * HALF, HALF)], src_v)
        pltpu.sync_copy(dstr_hbm.at[wid, pl.ds(hh * HALF, HALF)], dst_v)
        pltpu.async_copy(u_hbm.at[src_v.at[0]], gbuf0, gsem0)
        pltpu.async_copy(u_hbm.at[src_v.at[1]], gbuf1, gsem1)

        @pl.loop(0, HALF // 2)
        def _(i):
            j0 = 2 * i
            pltpu.make_async_copy(u_hbm.at[src_v.at[j0]], gbuf0, gsem0).wait()
            pltpu.async_copy(gbuf0, acc_sh.at[dst_v.at[j0]], ssem0, add=True)
            pltpu.make_async_copy(
                u_hbm.at[src_v.at[j0 + 1]], gbuf1, gsem1).wait()
            pltpu.async_copy(
                gbuf1, acc_sh.at[dst_v.at[j0 + 1]], ssem1, add=True)

            @pl.when(j0 + 2 < HALF)
            def _():
                pltpu.make_async_copy(
                    gbuf0, acc_sh.at[dst_v.at[j0]], ssem0).wait()
                pltpu.async_copy(u_hbm.at[src_v.at[j0 + 2]], gbuf0, gsem0)
                pltpu.make_async_copy(
                    gbuf1, acc_sh.at[dst_v.at[j0 + 1]], ssem1).wait()
                pltpu.async_copy(u_hbm.at[src_v.at[j0 + 3]], gbuf1, gsem1)

        pltpu.make_async_copy(
            gbuf0, acc_sh.at[dst_v.at[HALF - 2]], ssem0).wait()
        pltpu.make_async_copy(
            gbuf1, acc_sh.at[dst_v.at[HALF - 1]], ssem1).wait()

    plsc.subcore_barrier()
    _writeback_my_rows(acc_sh, gbuf0, p_hbm, c, base)


@functools.partial(
    pl.kernel,
    out_type=jax.ShapeDtypeStruct((SC_CORES, NP, H), _f32),
    mesh=_mesh,
    scratch_types=[
        pltpu.VMEM((CH, CHUNK), jnp.int32),
        pltpu.VMEM((CHUNK, H), _f32),
        pltpu.VMEM((CHUNK, H), _f32),
        pltpu.VMEM_SHARED((NP, H), _f32),
        pltpu.SemaphoreType.DMA,
    ],
)
def _sc_deg(dstr_hbm, orow_hbm, zrow_hbm, p_hbm, dst_v, obuf, wbuf, acc_sh,
            sem):
    c = lax.axis_index("c")
    s = lax.axis_index("s")
    wid = s * SC_CORES + c
    pltpu.sync_copy(dstr_hbm.at[wid], dst_v)
    pltpu.sync_copy(orow_hbm, obuf)
    base = s * ROWS_PT
    _zero_my_rows(zrow_hbm, wbuf, acc_sh, base)
    plsc.subcore_barrier()

    @pl.loop(0, CH)
    def _(j):
        pltpu.async_copy(obuf, acc_sh.at[dst_v.at[j]], sem, add=True)

    @pl.loop(0, CH)
    def _(j):
        pltpu.make_async_copy(obuf, acc_sh.at[dst_v.at[j]], sem).wait()

    plsc.subcore_barrier()
    _writeback_my_rows(acc_sh, wbuf, p_hbm, c, base)


def _tc_pre_body(pdeg_ref, snp_ref, rs_ref, rs2_ref, sc2_ref):
    deg = jnp.maximum(pdeg_ref[0] + pdeg_ref[1], 1.0)
    rs = lax.rsqrt(deg)
    rs_ref[...] = rs
    rs2_ref[...] = 1.0 / deg
    sc2_ref[...] = rs * snp_ref[...]


def _tc_mm0_body(h_ref, emb_ref, ws0_ref, rs_ref, x_ref, u_ref):
    hv = h_ref[...]
    iot = lax.broadcasted_iota(jnp.int32, (1, IN_DIM), 1)
    oh = (hv == iot).astype(_f32)
    x = jnp.dot(oh, emb_ref[...], preferred_element_type=_f32)
    x_ref[...] = x
    xw = (jnp.dot(x, ws0_ref[0], preferred_element_type=_f32)
          + jnp.dot(x, ws0_ref[1], preferred_element_type=_f32)
          + jnp.dot(x, ws0_ref[2], preferred_element_type=_f32))
    u_ref[pl.ds(0, N)] = rs_ref[pl.ds(0, N)] * xw
    u_ref[pl.ds(N, NP - N)] = jnp.zeros((NP - N, H), _f32)


def _tc_scale_body(p_ref, rs2_ref, u_ref):
    u_ref[...] = rs2_ref[...] * (p_ref[0] + p_ref[1])


def _bn_relu_res(p_ref, x_ref, sc2_ref, g_ref, b_ref):
    t = p_ref[0, pl.ds(0, N)] + p_ref[1, pl.ds(0, N)]
    v = sc2_ref[pl.ds(0, N)] * t
    mean = jnp.mean(v, axis=0, keepdims=True)
    var = jnp.mean((v - mean) ** 2, axis=0, keepdims=True)
    y = g_ref[...] * (v - mean) * lax.rsqrt(var + 1e-5) + b_ref[...]
    return x_ref[...] + jnp.maximum(y, 0.0)


def _tc_layer_body(p_ref, x_ref, sc2_ref, g_ref, b_ref, wsn_ref, rs_ref,
                   xn_ref, u_ref):
    xn = _bn_relu_res(p_ref, x_ref, sc2_ref, g_ref, b_ref)
    xn_ref[...] = xn
    xw = (jnp.dot(xn, wsn_ref[0], preferred_element_type=_f32)
          + jnp.dot(xn, wsn_ref[1], preferred_element_type=_f32)
          + jnp.dot(xn, wsn_ref[2], preferred_element_type=_f32))
    u_ref[pl.ds(0, N)] = rs_ref[pl.ds(0, N)] * xw
    u_ref[pl.ds(N, NP - N)] = jnp.zeros((NP - N, H), _f32)


def _tc_final_body(p_ref, x_ref, sc2_ref, g_ref, b_ref,
                   w1_ref, b1_ref, w2_ref, b2_ref, w3_ref, b3_ref, o_ref):
    xn = _bn_relu_res(p_ref, x_ref, sc2_ref, g_ref, b_ref)
    y = jnp.maximum(jnp.dot(xn, w1_ref[...], preferred_element_type=_f32)
                    + b1_ref[...], 0.0)
    y = jnp.maximum(jnp.dot(y, w2_ref[...], preferred_element_type=_f32)
                    + b2_ref[...], 0.0)
    o_ref[...] = jnp.dot(y, w3_ref[...], preferred_element_type=_f32) \
        + b3_ref[...]


_tc_pre = pl.pallas_call(
    _tc_pre_body,
    out_shape=[jax.ShapeDtypeStruct((NP, H), _f32)] * 3)

_tc_mm0 = pl.pallas_call(
    _tc_mm0_body,
    out_shape=[jax.ShapeDtypeStruct((N, H), _f32),
               jax.ShapeDtypeStruct((NP, H), _f32)])

_tc_scale = pl.pallas_call(
    _tc_scale_body,
    out_shape=jax.ShapeDtypeStruct((NP, H), _f32))

_tc_layer = pl.pallas_call(
    _tc_layer_body,
    out_shape=[jax.ShapeDtypeStruct((N, H), _f32),
               jax.ShapeDtypeStruct((NP, H), _f32)])

_tc_final = pl.pallas_call(
    _tc_final_body,
    out_shape=jax.ShapeDtypeStruct((N, 8), _f32))


def kernel(h, edge_index, e, snorm_n, snorm_e, emb, Ws, bn_gamma, bn_beta,
           W1, b1, W2, b2, W3, b3):
    del e, snorm_e
    src = edge_index[0].astype(jnp.int32)
    dst = edge_index[1].astype(jnp.int32)
    pad = jnp.full((EP - E,), N, jnp.int32)
    srcr = jnp.concatenate([src, pad]).reshape(NWORK, CH, CHUNK)
    dstr = jnp.concatenate([dst, pad]).reshape(NWORK, CH, CHUNK)
    zrow = jnp.zeros((CHUNK, H), _f32)
    orow = jnp.ones((CHUNK, H), _f32)
    snp = jnp.concatenate(
        [snorm_n.astype(_f32), jnp.zeros((NP - N, 1), _f32)], axis=0)
    h2 = h.astype(jnp.int32).reshape(N, 1)

    pdeg = _sc_deg(dstr, orow, zrow)
    rs, rs2, sc2 = _tc_pre(pdeg, snp)
    x, u = _tc_mm0(h2, emb, Ws[0], rs)
    out = None
    for l in range(L):
        p = None
        for k in range(K):
            p = _sc_prop(u, srcr, dstr, zrow)
            if k < K - 1:
                u = _tc_scale(p, rs2)
        if l < L - 1:
            x, u = _tc_layer(p, x, sc2, bn_gamma[l][None], bn_beta[l][None],
                             Ws[l + 1], rs)
        else:
            out = _tc_final(p, x, sc2, bn_gamma[l][None], bn_beta[l][None],
                            W1, b1[None], W2, b2[None], W3, b3[None])
    return out


# speedup vs baseline: 1.7227x; 1.7227x over previous
import functools

import jax
import jax.numpy as jnp
from jax import lax
from jax.experimental import pallas as pl
from jax.experimental.pallas import tpu as pltpu
from jax.experimental.pallas import tpu_sc as plsc

N = 10000
E = 320000
H = 128
HH = H // 2
L = 4
NF = 3
K = 3
IN_DIM = 128

SC_CORES = 2
SC_TILES = 16
NWORK = SC_CORES * SC_TILES
CHUNK = 128
CH = 80
EPT = CH * CHUNK
EP = NWORK * EPT
NP = 10112
ROWS_PT = NP // SC_TILES
NFULL = ROWS_PT // CHUNK
NREM = ROWS_PT - NFULL * CHUNK

_f32 = jnp.float32
_mesh = plsc.VectorSubcoreMesh(
    core_axis_name="c", subcore_axis_name="s",
    num_cores=SC_CORES, num_subcores=SC_TILES)


def _rows_hop(src_hbm_or_sh, dst_hbm_or_sh, buf, base):
    @pl.loop(0, NFULL)
    def _(i):
        pltpu.sync_copy(src_hbm_or_sh.at[pl.ds(base + i * CHUNK, CHUNK)], buf)
        pltpu.sync_copy(buf, dst_hbm_or_sh.at[pl.ds(base + i * CHUNK, CHUNK)])

    rem0 = base + NFULL * CHUNK
    pltpu.sync_copy(src_hbm_or_sh.at[pl.ds(rem0, NREM)],
                    buf.at[pl.ds(0, NREM)])
    pltpu.sync_copy(buf.at[pl.ds(0, NREM)],
                    dst_hbm_or_sh.at[pl.ds(rem0, NREM)])


def _zero_my_rows(zrow_hbm, buf, acc_sh, base):
    pltpu.sync_copy(zrow_hbm, buf)

    @pl.loop(0, NFULL)
    def _(i):
        pltpu.sync_copy(buf, acc_sh.at[pl.ds(base + i * CHUNK, CHUNK)])

    pltpu.sync_copy(buf.at[pl.ds(0, NREM)],
                    acc_sh.at[pl.ds(base + NFULL * CHUNK, NREM)])


HALF = CH // 2


@functools.partial(
    pl.kernel,
    out_type=jax.ShapeDtypeStruct((2, SC_CORES, NP, HH), _f32),
    mesh=_mesh,
    compiler_params=pltpu.CompilerParams(use_tc_tiling_on_sc=False),
    scratch_types=[
        pltpu.VMEM((HALF, CHUNK), jnp.int32),
        pltpu.VMEM((HALF, CHUNK), jnp.int32),
        pltpu.VMEM((CHUNK, HH), _f32),
        pltpu.VMEM((CHUNK, HH), _f32),
        pltpu.VMEM_SHARED((NP, HH), _f32),
        pltpu.VMEM_SHARED((NP, HH), _f32),
        pltpu.SemaphoreType.DMA,
        pltpu.SemaphoreType.DMA,
        pltpu.SemaphoreType.DMA,
        pltpu.SemaphoreType.DMA,
    ],
)
def _sc_prop(u2_hbm, srcr_hbm, dstr_hbm, zrow_hbm, p2_hbm,
             src_v, dst_v, gbuf0, gbuf1, u_sh, acc_sh,
             gsem0, gsem1, ssem0, ssem1):
    c = lax.axis_index("c")
    s = lax.axis_index("s")
    wid = s * SC_CORES + c
    base = s * ROWS_PT

    for hc in range(2):
        _rows_hop(u2_hbm.at[hc], u_sh, gbuf0, base)
        _zero_my_rows(zrow_hbm, gbuf0, acc_sh, base)
        plsc.subcore_barrier()

        for hh in range(CH // HALF):
            pltpu.sync_copy(srcr_hbm.at[wid, pl.ds(hh * HALF, HALF)], src_v)
            pltpu.sync_copy(dstr_hbm.at[wid, pl.ds(hh * HALF, HALF)], dst_v)
            pltpu.async_copy(u_sh.at[src_v.at[0]], gbuf0, gsem0)
            pltpu.async_copy(u_sh.at[src_v.at[1]], gbuf1, gsem1)

            @pl.loop(0, HALF // 2)
            def _(i):
                j0 = 2 * i
                pltpu.make_async_copy(
                    u_sh.at[src_v.at[j0]], gbuf0, gsem0).wait()
                pltpu.async_copy(
                    gbuf0, acc_sh.at[dst_v.at[j0]], ssem0, add=True)
                pltpu.make_async_copy(
                    u_sh.at[src_v.at[j0 + 1]], gbuf1, gsem1).wait()
                pltpu.async_copy(
                    gbuf1, acc_sh.at[dst_v.at[j0 + 1]], ssem1, add=True)

                @pl.when(j0 + 2 < HALF)
                def _():
                    pltpu.make_async_copy(
                        gbuf0, acc_sh.at[dst_v.at[j0]], ssem0).wait()
                    pltpu.async_copy(u_sh.at[src_v.at[j0 + 2]], gbuf0, gsem0)
                    pltpu.make_async_copy(
                        gbuf1, acc_sh.at[dst_v.at[j0 + 1]], ssem1).wait()
                    pltpu.async_copy(u_sh.at[src_v.at[j0 + 3]], gbuf1, gsem1)

            pltpu.make_async_copy(
                gbuf0, acc_sh.at[dst_v.at[HALF - 2]], ssem0).wait()
            pltpu.make_async_copy(
                gbuf1, acc_sh.at[dst_v.at[HALF - 1]], ssem1).wait()

        plsc.subcore_barrier()
        _rows_hop(acc_sh, p2_hbm.at[hc, c], gbuf0, base)


@functools.partial(
    pl.kernel,
    out_type=jax.ShapeDtypeStruct((SC_CORES, NP, HH), _f32),
    mesh=_mesh,
    compiler_params=pltpu.CompilerParams(use_tc_tiling_on_sc=False),
    scratch_types=[
        pltpu.VMEM((CH, CHUNK), jnp.int32),
        pltpu.VMEM((CHUNK, HH), _f32),
        pltpu.VMEM((CHUNK, HH), _f32),
        pltpu.VMEM_SHARED((NP, HH), _f32),
        pltpu.SemaphoreType.DMA,
    ],
)
def _sc_deg(dstr_hbm, orow_hbm, zrow_hbm, p_hbm, dst_v, obuf, wbuf, acc_sh,
            sem):
    c = lax.axis_index("c")
    s = lax.axis_index("s")
    wid = s * SC_CORES + c
    pltpu.sync_copy(dstr_hbm.at[wid], dst_v)
    pltpu.sync_copy(orow_hbm, obuf)
    base = s * ROWS_PT
    _zero_my_rows(zrow_hbm, wbuf, acc_sh, base)
    plsc.subcore_barrier()

    @pl.loop(0, CH)
    def _(j):
        pltpu.async_copy(obuf, acc_sh.at[dst_v.at[j]], sem, add=True)

    @pl.loop(0, CH)
    def _(j):
        pltpu.make_async_copy(obuf, acc_sh.at[dst_v.at[j]], sem).wait()

    plsc.subcore_barrier()
    _rows_hop(acc_sh, p_hbm.at[c], wbuf, base)


def _tc_pre_body(pdeg_ref, snp_ref, rs_ref, rs2_ref, sc2_ref):
    degh = jnp.maximum(pdeg_ref[0] + pdeg_ref[1], 1.0)
    rsh = lax.rsqrt(degh)
    rs = jnp.concatenate([rsh, rsh], axis=1)
    rs_ref[...] = rs
    rs2h = 1.0 / degh
    rs2_ref[...] = jnp.concatenate([rs2h, rs2h], axis=1)
    sc2_ref[...] = rs * snp_ref[...]


def _store_u_halves(u_ref, uval):
    u_ref[0, pl.ds(0, N)] = uval[:, :HH]
    u_ref[1, pl.ds(0, N)] = uval[:, HH:]
    zpad = jnp.zeros((NP - N, HH), _f32)
    u_ref[0, pl.ds(N, NP - N)] = zpad
    u_ref[1, pl.ds(N, NP - N)] = zpad


def _tc_mm0_body(h_ref, emb_ref, ws0_ref, rs_ref, x_ref, u_ref):
    hv = h_ref[...]
    iot = lax.broadcasted_iota(jnp.int32, (1, IN_DIM), 1)
    oh = (hv == iot).astype(_f32)
    x = jnp.dot(oh, emb_ref[...], preferred_element_type=_f32)
    x_ref[...] = x
    xw = (jnp.dot(x, ws0_ref[0], preferred_element_type=_f32)
          + jnp.dot(x, ws0_ref[1], preferred_element_type=_f32)
          + jnp.dot(x, ws0_ref[2], preferred_element_type=_f32))
    _store_u_halves(u_ref, rs_ref[pl.ds(0, N)] * xw)


def _tc_scale_body(p_ref, rs2_ref, u_ref):
    r = rs2_ref[...]
    u_ref[0] = r[:, :HH] * (p_ref[0, 0] + p_ref[0, 1])
    u_ref[1] = r[:, HH:] * (p_ref[1, 0] + p_ref[1, 1])


def _bn_relu_res(p_ref, x_ref, sc2_ref, g_ref, b_ref):
    t = jnp.concatenate(
        [p_ref[0, 0, pl.ds(0, N)] + p_ref[0, 1, pl.ds(0, N)],
         p_ref[1, 0, pl.ds(0, N)] + p_ref[1, 1, pl.ds(0, N)]], axis=1)
    v = sc2_ref[pl.ds(0, N)] * t
    mean = jnp.mean(v, axis=0, keepdims=True)
    var = jnp.mean((v - mean) ** 2, axis=0, keepdims=True)
    y = g_ref[...] * (v - mean) * lax.rsqrt(var + 1e-5) + b_ref[...]
    return x_ref[...] + jnp.maximum(y, 0.0)


def _tc_layer_body(p_ref, x_ref, sc2_ref, g_ref, b_ref, wsn_ref, rs_ref,
                   xn_ref, u_ref):
    xn = _bn_relu_res(p_ref, x_ref, sc2_ref, g_ref, b_ref)
    xn_ref[...] = xn
    xw = (jnp.dot(xn, wsn_ref[0], preferred_element_type=_f32)
          + jnp.dot(xn, wsn_ref[1], preferred_element_type=_f32)
          + jnp.dot(xn, wsn_ref[2], preferred_element_type=_f32))
    _store_u_halves(u_ref, rs_ref[pl.ds(0, N)] * xw)


def _tc_final_body(p_ref, x_ref, sc2_ref, g_ref, b_ref,
                   w1_ref, b1_ref, w2_ref, b2_ref, w3_ref, b3_ref, o_ref):
    xn = _bn_relu_res(p_ref, x_ref, sc2_ref, g_ref, b_ref)
    y = jnp.maximum(jnp.dot(xn, w1_ref[...], preferred_element_type=_f32)
                    + b1_ref[...], 0.0)
    y = jnp.maximum(jnp.dot(y, w2_ref[...], preferred_element_type=_f32)
                    + b2_ref[...], 0.0)
    o_ref[...] = jnp.dot(y, w3_ref[...], preferred_element_type=_f32) \
        + b3_ref[...]


_tc_pre = pl.pallas_call(
    _tc_pre_body,
    out_shape=[jax.ShapeDtypeStruct((NP, H), _f32)] * 3)

_tc_mm0 = pl.pallas_call(
    _tc_mm0_body,
    out_shape=[jax.ShapeDtypeStruct((N, H), _f32),
               jax.ShapeDtypeStruct((2, NP, HH), _f32)])

_tc_scale = pl.pallas_call(
    _tc_scale_body,
    out_shape=jax.ShapeDtypeStruct((2, NP, HH), _f32))

_tc_layer = pl.pallas_call(
    _tc_layer_body,
    out_shape=[jax.ShapeDtypeStruct((N, H), _f32),
               jax.ShapeDtypeStruct((2, NP, HH), _f32)])

_tc_final = pl.pallas_call(
    _tc_final_body,
    out_shape=jax.ShapeDtypeStruct((N, 8), _f32))


def kernel(h, edge_index, e, snorm_n, snorm_e, emb, Ws, bn_gamma, bn_beta,
           W1, b1, W2, b2, W3, b3):
    del e, snorm_e
    src = edge_index[0].astype(jnp.int32)
    dst = edge_index[1].astype(jnp.int32)
    pad = jnp.full((EP - E,), N, jnp.int32)
    srcr = jnp.concatenate([src, pad]).reshape(NWORK, CH, CHUNK)
    dstr = jnp.concatenate([dst, pad]).reshape(NWORK, CH, CHUNK)
    zrow = jnp.zeros((CHUNK, HH), _f32)
    orow = jnp.ones((CHUNK, HH), _f32)
    snp = jnp.concatenate(
        [snorm_n.astype(_f32), jnp.zeros((NP - N, 1), _f32)], axis=0)
    h2 = h.astype(jnp.int32).reshape(N, 1)

    pdeg = _sc_deg(dstr, orow, zrow)
    rs, rs2, sc2 = _tc_pre(pdeg, snp)
    x, u = _tc_mm0(h2, emb, Ws[0], rs)
    out = None
    for l in range(L):
        p = None
        for k in range(K):
            p = _sc_prop(u, srcr, dstr, zrow)
            if k < K - 1:
                u = _tc_scale(p, rs2)
        if l < L - 1:
            x, u = _tc_layer(p, x, sc2, bn_gamma[l][None], bn_beta[l][None],
                             Ws[l + 1], rs)
        else:
            out = _tc_final(p, x, sc2, bn_gamma[l][None], bn_beta[l][None],
                            W1, b1[None], W2, b2[None], W3, b3[None])
    return out


# speedup vs baseline: 1.7905x; 1.0394x over previous
import functools

import jax
import jax.numpy as jnp
from jax import lax
from jax.experimental import pallas as pl
from jax.experimental.pallas import tpu as pltpu
from jax.experimental.pallas import tpu_sc as plsc

N = 10000
E = 320000
H = 128
HH = H // 2
L = 4
NF = 3
K = 3
IN_DIM = 128

SC_CORES = 2
SC_TILES = 16
NWORK = SC_CORES * SC_TILES
CHUNK = 128
CH = 80
EPT = CH * CHUNK
EP = NWORK * EPT
NP = 10112
ROWS_PT = NP // SC_TILES
NFULL = ROWS_PT // CHUNK
NREM = ROWS_PT - NFULL * CHUNK

_f32 = jnp.float32
_mesh = plsc.VectorSubcoreMesh(
    core_axis_name="c", subcore_axis_name="s",
    num_cores=SC_CORES, num_subcores=SC_TILES)


HALF = CH // 2


NBUF = 4


@functools.partial(
    pl.kernel,
    out_type=jax.ShapeDtypeStruct((2, SC_CORES, NP, HH), _f32),
    mesh=_mesh,
    compiler_params=pltpu.CompilerParams(use_tc_tiling_on_sc=False),
    scratch_types=[
        pltpu.VMEM((HALF, CHUNK), jnp.int32),
        pltpu.VMEM((HALF, CHUNK), jnp.int32),
        [pltpu.VMEM((CHUNK, HH), _f32)] * NBUF,
        pltpu.VMEM_SHARED((NP, HH), _f32),
        pltpu.VMEM_SHARED((NP, HH), _f32),
        [pltpu.SemaphoreType.DMA] * NBUF,
        [pltpu.SemaphoreType.DMA] * NBUF,
    ],
)
def _sc_prop(u2_hbm, srcr_hbm, dstr_hbm, zrows_hbm, p2_hbm,
             src_v, dst_v, gbufs, u_sh, acc_sh, gsems, ssems):
    c = lax.axis_index("c")
    s = lax.axis_index("s")
    wid = s * SC_CORES + c
    base = s * ROWS_PT

    for hc in range(2):
        pltpu.sync_copy(u2_hbm.at[hc, pl.ds(base, ROWS_PT)],
                        u_sh.at[pl.ds(base, ROWS_PT)])
        pltpu.sync_copy(zrows_hbm.at[pl.ds(base, ROWS_PT)],
                        acc_sh.at[pl.ds(base, ROWS_PT)])
        plsc.subcore_barrier()

        for hh in range(CH // HALF):
            pltpu.sync_copy(srcr_hbm.at[wid, pl.ds(hh * HALF, HALF)], src_v)
            pltpu.sync_copy(dstr_hbm.at[wid, pl.ds(hh * HALF, HALF)], dst_v)
            for b in range(NBUF):
                pltpu.async_copy(u_sh.at[src_v.at[b]], gbufs[b], gsems[b])

            @pl.loop(0, HALF // NBUF)
            def _(i):
                j0 = NBUF * i
                for b in range(NBUF):
                    pltpu.make_async_copy(
                        u_sh.at[src_v.at[j0 + b]], gbufs[b], gsems[b]).wait()
                    pltpu.async_copy(gbufs[b], acc_sh.at[dst_v.at[j0 + b]],
                                     ssems[b], add=True)
                for b in range(NBUF):
                    @pl.when(j0 + NBUF + b < HALF)
                    def _():
                        pltpu.make_async_copy(
                            gbufs[b], acc_sh.at[dst_v.at[j0 + b]],
                            ssems[b]).wait()
                        pltpu.async_copy(u_sh.at[src_v.at[j0 + NBUF + b]],
                                         gbufs[b], gsems[b])

            for b in range(NBUF):
                pltpu.make_async_copy(
                    gbufs[b], acc_sh.at[dst_v.at[HALF - NBUF + b]],
                    ssems[b]).wait()

        plsc.subcore_barrier()
        pltpu.sync_copy(acc_sh.at[pl.ds(base, ROWS_PT)],
                        p2_hbm.at[hc, c, pl.ds(base, ROWS_PT)])


@functools.partial(
    pl.kernel,
    out_type=jax.ShapeDtypeStruct((SC_CORES, NP, HH), _f32),
    mesh=_mesh,
    compiler_params=pltpu.CompilerParams(use_tc_tiling_on_sc=False),
    scratch_types=[
        pltpu.VMEM((CH, CHUNK), jnp.int32),
        pltpu.VMEM((CHUNK, HH), _f32),
        pltpu.VMEM_SHARED((NP, HH), _f32),
        pltpu.SemaphoreType.DMA,
    ],
)
def _sc_deg(dstr_hbm, orow_hbm, zrows_hbm, p_hbm, dst_v, obuf, acc_sh,
            sem):
    c = lax.axis_index("c")
    s = lax.axis_index("s")
    wid = s * SC_CORES + c
    pltpu.sync_copy(dstr_hbm.at[wid], dst_v)
    pltpu.sync_copy(orow_hbm, obuf)
    base = s * ROWS_PT
    pltpu.sync_copy(zrows_hbm.at[pl.ds(base, ROWS_PT)],
                    acc_sh.at[pl.ds(base, ROWS_PT)])
    plsc.subcore_barrier()

    @pl.loop(0, CH)
    def _(j):
        pltpu.async_copy(obuf, acc_sh.at[dst_v.at[j]], sem, add=True)

    @pl.loop(0, CH)
    def _(j):
        pltpu.make_async_copy(obuf, acc_sh.at[dst_v.at[j]], sem).wait()

    plsc.subcore_barrier()
    pltpu.sync_copy(acc_sh.at[pl.ds(base, ROWS_PT)],
                    p_hbm.at[c, pl.ds(base, ROWS_PT)])


def _tc_pre_body(pdeg_ref, snp_ref, rs_ref, rs2_ref, sc2_ref):
    degh = jnp.maximum(pdeg_ref[0] + pdeg_ref[1], 1.0)
    rsh = lax.rsqrt(degh)
    rs = jnp.concatenate([rsh, rsh], axis=1)
    rs_ref[...] = rs
    rs2h = 1.0 / degh
    rs2_ref[...] = jnp.concatenate([rs2h, rs2h], axis=1)
    sc2_ref[...] = rs * snp_ref[...]


def _store_u_halves(u_ref, uval):
    u_ref[0, pl.ds(0, N)] = uval[:, :HH]
    u_ref[1, pl.ds(0, N)] = uval[:, HH:]
    zpad = jnp.zeros((NP - N, HH), _f32)
    u_ref[0, pl.ds(N, NP - N)] = zpad
    u_ref[1, pl.ds(N, NP - N)] = zpad


def _tc_mm0_body(h_ref, emb_ref, ws0_ref, rs_ref, x_ref, u_ref):
    hv = h_ref[...]
    iot = lax.broadcasted_iota(jnp.int32, (1, IN_DIM), 1)
    oh = (hv == iot).astype(_f32)
    x = jnp.dot(oh, emb_ref[...], preferred_element_type=_f32)
    x_ref[...] = x
    xw = (jnp.dot(x, ws0_ref[0], preferred_element_type=_f32)
          + jnp.dot(x, ws0_ref[1], preferred_element_type=_f32)
          + jnp.dot(x, ws0_ref[2], preferred_element_type=_f32))
    _store_u_halves(u_ref, rs_ref[pl.ds(0, N)] * xw)


def _tc_scale_body(p_ref, rs2_ref, u_ref):
    r = rs2_ref[...]
    u_ref[0] = r[:, :HH] * (p_ref[0, 0] + p_ref[0, 1])
    u_ref[1] = r[:, HH:] * (p_ref[1, 0] + p_ref[1, 1])


def _bn_relu_res(p_ref, x_ref, sc2_ref, g_ref, b_ref):
    t = jnp.concatenate(
        [p_ref[0, 0, pl.ds(0, N)] + p_ref[0, 1, pl.ds(0, N)],
         p_ref[1, 0, pl.ds(0, N)] + p_ref[1, 1, pl.ds(0, N)]], axis=1)
    v = sc2_ref[pl.ds(0, N)] * t
    mean = jnp.mean(v, axis=0, keepdims=True)
    var = jnp.mean((v - mean) ** 2, axis=0, keepdims=True)
    y = g_ref[...] * (v - mean) * lax.rsqrt(var + 1e-5) + b_ref[...]
    return x_ref[...] + jnp.maximum(y, 0.0)


def _tc_layer_body(p_ref, x_ref, sc2_ref, g_ref, b_ref, wsn_ref, rs_ref,
                   xn_ref, u_ref):
    xn = _bn_relu_res(p_ref, x_ref, sc2_ref, g_ref, b_ref)
    xn_ref[...] = xn
    xw = (jnp.dot(xn, wsn_ref[0], preferred_element_type=_f32)
          + jnp.dot(xn, wsn_ref[1], preferred_element_type=_f32)
          + jnp.dot(xn, wsn_ref[2], preferred_element_type=_f32))
    _store_u_halves(u_ref, rs_ref[pl.ds(0, N)] * xw)


def _tc_final_body(p_ref, x_ref, sc2_ref, g_ref, b_ref,
                   w1_ref, b1_ref, w2_ref, b2_ref, w3_ref, b3_ref, o_ref):
    xn = _bn_relu_res(p_ref, x_ref, sc2_ref, g_ref, b_ref)
    y = jnp.maximum(jnp.dot(xn, w1_ref[...], preferred_element_type=_f32)
                    + b1_ref[...], 0.0)
    y = jnp.maximum(jnp.dot(y, w2_ref[...], preferred_element_type=_f32)
                    + b2_ref[...], 0.0)
    o_ref[...] = jnp.dot(y, w3_ref[...], preferred_element_type=_f32) \
        + b3_ref[...]


_tc_pre = pl.pallas_call(
    _tc_pre_body,
    out_shape=[jax.ShapeDtypeStruct((NP, H), _f32)] * 3)

_tc_mm0 = pl.pallas_call(
    _tc_mm0_body,
    out_shape=[jax.ShapeDtypeStruct((N, H), _f32),
               jax.ShapeDtypeStruct((2, NP, HH), _f32)])

_tc_scale = pl.pallas_call(
    _tc_scale_body,
    out_shape=jax.ShapeDtypeStruct((2, NP, HH), _f32))

_tc_layer = pl.pallas_call(
    _tc_layer_body,
    out_shape=[jax.ShapeDtypeStruct((N, H), _f32),
               jax.ShapeDtypeStruct((2, NP, HH), _f32)])

_tc_final = pl.pallas_call(
    _tc_final_body,
    out_shape=jax.ShapeDtypeStruct((N, 8), _f32))


def kernel(h, edge_index, e, snorm_n, snorm_e, emb, Ws, bn_gamma, bn_beta,
           W1, b1, W2, b2, W3, b3):
    del e, snorm_e
    src = edge_index[0].astype(jnp.int32)
    dst = edge_index[1].astype(jnp.int32)
    pad = jnp.full((EP - E,), N, jnp.int32)
    srcr = jnp.concatenate([src, pad]).reshape(NWORK, CH, CHUNK)
    dstr = jnp.concatenate([dst, pad]).reshape(NWORK, CH, CHUNK)
    zrows = jnp.zeros((NP, HH), _f32)
    orow = jnp.ones((CHUNK, HH), _f32)
    snp = jnp.concatenate(
        [snorm_n.astype(_f32), jnp.zeros((NP - N, 1), _f32)], axis=0)
    h2 = h.astype(jnp.int32).reshape(N, 1)

    pdeg = _sc_deg(dstr, orow, zrows)
    rs, rs2, sc2 = _tc_pre(pdeg, snp)
    x, u = _tc_mm0(h2, emb, Ws[0], rs)
    out = None
    for l in range(L):
        p = None
        for k in range(K):
            p = _sc_prop(u, srcr, dstr, zrows)
            if k < K - 1:
                u = _tc_scale(p, rs2)
        if l < L - 1:
            x, u = _tc_layer(p, x, sc2, bn_gamma[l][None], bn_beta[l][None],
                             Ws[l + 1], rs)
        else:
            out = _tc_final(p, x, sc2, bn_gamma[l][None], bn_beta[l][None],
                            W1, b1[None], W2, b2[None], W3, b3[None])
    return out
